# Initial kernel scaffold; baseline (speedup 1.0000x reference)
import jax
import jax.numpy as jnp
from jax.experimental import pallas as pl


def kernel(z, rand_inds):
    raise NotImplementedError("write your pallas kernel here")


# speedup vs baseline: 1.3311x; 1.3311x over previous
import functools

import jax
import jax.numpy as jnp
from jax import lax
from jax.experimental import pallas as pl
from jax.experimental.pallas import tpu as pltpu
from jax.experimental.pallas import tpu_sc as plsc

FUDGE = 1e-07

B = 320000
D = 128
NC = 2
NS = 16
L = 16
NW = NC * NS
BPW = B // NW
G = 80
NSTEPS = BPW // G
NQ = G // L


def _dots_for_chunk(s_ref, d_ref, out_ref, out_base):
    row_iota = lax.iota(jnp.int32, L)
    one = jnp.full((L,), 1.0, dtype=jnp.float32)
    scale = jnp.full((L,), 1.0 - 2.0 * FUDGE, dtype=jnp.float32)
    fudge = jnp.full((L,), FUDGE, dtype=jnp.float32)
    for q in range(NQ):
        rows = row_iota + (q * L)

        def dbody(t, acc):
            for j in range(4):
                cols = jnp.full((L,), t * 4 + j, dtype=jnp.int32)
                sv = plsc.load_gather(s_ref, [rows, cols])
                dv = plsc.load_gather(d_ref, [rows, cols])
                acc = acc + sv * dv
            return acc

        acc = lax.fori_loop(
            0, D // 4, dbody, jnp.zeros((L,), dtype=jnp.float32)
        )
        sig = one / (one + jnp.exp(-acc))
        out_ref[pl.ds(out_base + q * L, L)] = (sig + fudge) * scale


def _decode_kernel(z_hbm, sidx_hbm, didx_hbm, out_hbm,
                   sidx_v, didx_v, sA, dA, sB, dB, out_v,
                   sem_sa, sem_da, sem_sb, sem_db):
    wid = lax.axis_index("s") * NC + lax.axis_index("c")

    pltpu.sync_copy(sidx_hbm.at[wid], sidx_v)
    pltpu.sync_copy(didx_hbm.at[wid], didx_v)

    def start(g, s_buf, d_buf, s_sem, d_sem):
        pltpu.make_async_copy(z_hbm.at[sidx_v.at[g]], s_buf, s_sem).start()
        pltpu.make_async_copy(z_hbm.at[didx_v.at[g]], d_buf, d_sem).start()

    def wait(g, s_buf, d_buf, s_sem, d_sem):
        pltpu.make_async_copy(z_hbm.at[sidx_v.at[g]], s_buf, s_sem).wait()
        pltpu.make_async_copy(z_hbm.at[didx_v.at[g]], d_buf, d_sem).wait()

    start(0, sA, dA, sem_sa, sem_da)

    def gbody(t, carry):
        gA = t * 2
        gB = gA + 1

        @pl.when(gB < NSTEPS)
        def _():
            start(gB, sB, dB, sem_sb, sem_db)

        wait(gA, sA, dA, sem_sa, sem_da)
        _dots_for_chunk(sA, dA, out_v, gA * G)

        @pl.when(gB < NSTEPS)
        def _():
            @pl.when(gB + 1 < NSTEPS)
            def _():
                start(gB + 1, sA, dA, sem_sa, sem_da)

            wait(gB, sB, dB, sem_sb, sem_db)
            _dots_for_chunk(sB, dB, out_v, gB * G)

        return carry

    lax.fori_loop(0, (NSTEPS + 1) // 2, gbody, 0)

    pltpu.sync_copy(out_v, out_hbm.at[wid])


@jax.jit
def _decode(z, sidx, didx):
    mesh = plsc.VectorSubcoreMesh(
        core_axis_name="c", subcore_axis_name="s",
        num_cores=NC, num_subcores=NS,
    )
    f = pl.kernel(
        _decode_kernel,
        out_type=jax.ShapeDtypeStruct((NW, BPW), jnp.float32),
        mesh=mesh,
        scratch_types=[
            pltpu.VMEM((NSTEPS, G), jnp.int32),
            pltpu.VMEM((NSTEPS, G), jnp.int32),
            pltpu.VMEM((G, D), jnp.float32),
            pltpu.VMEM((G, D), jnp.float32),
            pltpu.VMEM((G, D), jnp.float32),
            pltpu.VMEM((G, D), jnp.float32),
            pltpu.VMEM((BPW,), jnp.float32),
            pltpu.SemaphoreType.DMA,
            pltpu.SemaphoreType.DMA,
            pltpu.SemaphoreType.DMA,
            pltpu.SemaphoreType.DMA,
        ],
        compiler_params=pltpu.CompilerParams(needs_layout_passes=False),
    )
    return f(z, sidx, didx)


def kernel(z, rand_inds):
    ri = rand_inds.astype(jnp.int32)
    sidx = ri[0].reshape(NW, NSTEPS, G)
    didx = ri[1].reshape(NW, NSTEPS, G)
    out = _decode(z, sidx, didx)
    return out.reshape(B)
